# baseline (speedup 1.0000x reference)
import functools

import jax
import jax.numpy as jnp
from jax import lax
from jax.experimental import pallas as pl
from jax.experimental.pallas import tpu as pltpu
from jax.experimental.pallas import tpu_sc as plsc

_B, _L, _D = 16384, 200, 32
_NC, _NS = 2, 16
_NW = _NC * _NS
_RPW = _B // _NW
_R = 4
_NCH = _RPW // _R
_G = _L * _D // 128

_mesh = plsc.VectorSubcoreMesh(core_axis_name="c", subcore_axis_name="s")


@functools.partial(
    pl.kernel,
    out_type=jax.ShapeDtypeStruct((_B // 4, 128), jnp.float32),
    mesh=_mesh,
    scratch_types=[
        pltpu.VMEM((2, _R * _G, 128), jnp.float32),
        pltpu.VMEM((2, _R, _L), jnp.float32),
        pltpu.VMEM((_RPW // 4, 128), jnp.float32),
        pltpu.SemaphoreType.DMA,
        pltpu.SemaphoreType.DMA,
    ],
)
def _sc_masked_sum(x_hbm, m_hbm, out_hbm, xbuf, mbuf, acc, semx, semm):
    w = lax.axis_index("s") * _NC + lax.axis_index("c")
    base = w * _RPW

    def start(c, slot):
        pltpu.async_copy(
            x_hbm.at[pl.ds((base + c * _R) * _G, _R * _G)], xbuf.at[slot], semx)
        pltpu.async_copy(
            m_hbm.at[pl.ds(base + c * _R, _R)], mbuf.at[slot], semm)

    def wait(slot):
        pltpu.make_async_copy(
            x_hbm.at[pl.ds(0, _R * _G)], xbuf.at[slot], semx).wait()
        pltpu.make_async_copy(
            m_hbm.at[pl.ds(0, _R)], mbuf.at[slot], semm).wait()

    def compute(c, slot):
        for r in range(_R):
            def gbody(i, carry, _slot=slot, _r=r):
                a0, a1 = carry
                mv16 = mbuf[_slot, _r, pl.ds(i * 16, 16)]
                for u in range(16):
                    g = _r * _G + i * 4 + u // 4
                    lane = (u % 4) * 32
                    a0 = a0 + mv16[u] * xbuf[_slot, g, pl.ds(lane, 16)]
                    a1 = a1 + mv16[u] * xbuf[_slot, g, pl.ds(lane + 16, 16)]
                return a0, a1

            z = jnp.zeros((16,), jnp.float32)
            a0, a1 = lax.fori_loop(0, _L // 16, gbody, (z, z))
            mv16 = mbuf[slot, r, pl.ds(_L - 16, 16)]
            for t in range(8):
                g = r * _G + 48 + t // 4
                lane = (t % 4) * 32
                a0 = a0 + mv16[t + 8] * xbuf[slot, g, pl.ds(lane, 16)]
                a1 = a1 + mv16[t + 8] * xbuf[slot, g, pl.ds(lane + 16, 16)]
            acc[c, pl.ds(r * 32, 16)] = a0
            acc[c, pl.ds(r * 32 + 16, 16)] = a1

    start(0, 0)
    start(1, 1)

    def chunk_pair(j, _):
        c0 = 2 * j
        wait(0)
        compute(c0, 0)

        @pl.when(j < _NCH // 2 - 1)
        def _():
            start(c0 + 2, 0)

        wait(1)
        compute(c0 + 1, 1)

        @pl.when(j < _NCH // 2 - 1)
        def _():
            start(c0 + 3, 1)

        return 0

    lax.fori_loop(0, _NCH // 2, chunk_pair, 0)
    pltpu.sync_copy(acc, out_hbm.at[pl.ds(w * (_RPW // 4), _RPW // 4)])


def kernel(inputs, mask):
    x2 = jnp.reshape(inputs, (_B * _G, 128))
    m32 = mask.astype(jnp.float32)
    out = _sc_masked_sum(x2, m32)
    return jnp.reshape(out, (_B, _D))


# speedup vs baseline: 1.0113x; 1.0113x over previous
import jax
import jax.numpy as jnp
from jax.experimental import pallas as pl
from jax.experimental.pallas import tpu as pltpu

_B, _L, _D = 16384, 200, 32
_BB = 64


def _body(x_ref, m_ref, o_ref):
    x = x_ref[...]
    m = m_ref[...]
    o_ref[...] = jnp.sum(x * m[:, :, None], axis=1)


def kernel(inputs, mask):
    m32 = mask.astype(jnp.float32)
    return pl.pallas_call(
        _body,
        grid=(_B // _BB,),
        in_specs=[
            pl.BlockSpec((_BB, _L, _D), lambda i: (i, 0, 0)),
            pl.BlockSpec((_BB, _L), lambda i: (i, 0)),
        ],
        out_specs=pl.BlockSpec((_BB, _D), lambda i: (i, 0)),
        out_shape=jax.ShapeDtypeStruct((_B, _D), jnp.float32),
        compiler_params=pltpu.CompilerParams(
            dimension_semantics=("arbitrary",),
        ),
    )(inputs, m32)


# speedup vs baseline: 1.3174x; 1.3026x over previous
import jax
import jax.numpy as jnp
from jax.experimental import pallas as pl
from jax.experimental.pallas import tpu as pltpu

_B, _L, _D = 16384, 200, 32
_BB = 256
_G = 50


def _body(x_ref, m_ref, o_ref):
    x = x_ref[...]
    m = m_ref[...]
    acc = jnp.sum(x[:, :, 0:32] * m[:, :, 0:1], axis=1)
    acc += jnp.sum(x[:, :, 32:64] * m[:, :, 1:2], axis=1)
    acc += jnp.sum(x[:, :, 64:96] * m[:, :, 2:3], axis=1)
    acc += jnp.sum(x[:, :, 96:128] * m[:, :, 3:4], axis=1)
    o_ref[...] = acc


def kernel(inputs, mask):
    x3 = jnp.reshape(inputs, (_B, _G, 128))
    m3 = jnp.reshape(mask.astype(jnp.float32), (_B, _G, 4))
    return pl.pallas_call(
        _body,
        grid=(_B // _BB,),
        in_specs=[
            pl.BlockSpec((_BB, _G, 128), lambda i: (i, 0, 0)),
            pl.BlockSpec((_BB, _G, 4), lambda i: (i, 0, 0)),
        ],
        out_specs=pl.BlockSpec((_BB, _D), lambda i: (i, 0)),
        out_shape=jax.ShapeDtypeStruct((_B, _D), jnp.float32),
        compiler_params=pltpu.CompilerParams(
            dimension_semantics=("arbitrary",),
        ),
    )(x3, m3)


# speedup vs baseline: 2.2565x; 1.7129x over previous
import jax
import jax.numpy as jnp
from jax.experimental import pallas as pl
from jax.experimental.pallas import tpu as pltpu

_B, _L, _D = 16384, 200, 32
_BB = 128
_W = _L * _D
_WA = 128 * _D
_WB = _W - _WA


def _body(x_ref, m_ref, o_ref):
    m = m_ref[...]
    idxa = jnp.broadcast_to(
        (jnp.arange(_WA, dtype=jnp.int32) // _D)[None, :], (_BB, _WA))
    mea = jnp.take_along_axis(m[:, 0:128], idxa, axis=1)
    idxb = jnp.broadcast_to(
        (jnp.arange(_WB, dtype=jnp.int32) // _D)[None, :], (_BB, _WB))
    meb = jnp.take_along_axis(m[:, 128:256], idxb, axis=1)
    x = x_ref[...]
    xa = x[:, 0:_WA] * mea
    xb = x[:, _WA:_W] * meb
    a0 = xa[:, 0:128]
    a1 = xa[:, 128:256]
    a2 = xa[:, 256:384]
    a3 = xa[:, 384:512]
    for k in range(4, 32, 4):
        a0 = a0 + xa[:, 128 * k:128 * (k + 1)]
        a1 = a1 + xa[:, 128 * (k + 1):128 * (k + 2)]
        a2 = a2 + xa[:, 128 * (k + 2):128 * (k + 3)]
        a3 = a3 + xa[:, 128 * (k + 3):128 * (k + 4)]
    for k in range(0, 16, 4):
        a0 = a0 + xb[:, 128 * k:128 * (k + 1)]
        a1 = a1 + xb[:, 128 * (k + 1):128 * (k + 2)]
        a2 = a2 + xb[:, 128 * (k + 2):128 * (k + 3)]
        a3 = a3 + xb[:, 128 * (k + 3):128 * (k + 4)]
    a0 = a0 + xb[:, 128 * 16:128 * 17]
    a1 = a1 + xb[:, 128 * 17:128 * 18]
    acc = (a0 + a1) + (a2 + a3)
    o_ref[...] = (acc[:, 0:32] + acc[:, 32:64]
                  + acc[:, 64:96] + acc[:, 96:128])


def kernel(inputs, mask):
    x2 = jnp.reshape(inputs, (_B, _W))
    m2 = jnp.pad(mask.astype(jnp.float32), ((0, 0), (0, 56)))
    return pl.pallas_call(
        _body,
        grid=(_B // _BB,),
        in_specs=[
            pl.BlockSpec((_BB, _W), lambda i: (i, 0)),
            pl.BlockSpec((_BB, 256), lambda i: (i, 0)),
        ],
        out_specs=pl.BlockSpec((_BB, _D), lambda i: (i, 0)),
        out_shape=jax.ShapeDtypeStruct((_B, _D), jnp.float32),
        compiler_params=pltpu.CompilerParams(
            dimension_semantics=("arbitrary",),
        ),
    )(x2, m2)


# speedup vs baseline: 2.6324x; 1.1665x over previous
import jax
import jax.numpy as jnp
from jax.experimental import pallas as pl
from jax.experimental.pallas import tpu as pltpu

_B, _L, _D = 16384, 200, 32
_BB = 128
_W = _L * _D
_WA = 128 * _D
_WB = _W - _WA


def _body(x_ref, m_ref, o_ref):
    m = m_ref[...]
    ma = m[:, 0:128]
    mb = m[:, 128:256]
    lane = jnp.arange(128, dtype=jnp.int32) // _D
    accs = [None, None, None, None]
    for k in range(50):
        src = ma if k < 32 else mb
        base = 4 * k if k < 32 else 4 * k - 128
        idx = jnp.broadcast_to((base + lane)[None, :], (_BB, 128))
        me = jnp.take_along_axis(src, idx, axis=1)
        term = x_ref[:, 128 * k:128 * (k + 1)] * me
        j = k % 4
        accs[j] = term if accs[j] is None else accs[j] + term
    acc = (accs[0] + accs[1]) + (accs[2] + accs[3])
    o_ref[...] = (acc[:, 0:32] + acc[:, 32:64]
                  + acc[:, 64:96] + acc[:, 96:128])


def kernel(inputs, mask):
    x2 = jnp.reshape(inputs, (_B, _W))
    m2 = jnp.pad(mask.astype(jnp.float32), ((0, 0), (0, 56)))
    return pl.pallas_call(
        _body,
        grid=(_B // _BB,),
        in_specs=[
            pl.BlockSpec((_BB, _W), lambda i: (i, 0)),
            pl.BlockSpec((_BB, 256), lambda i: (i, 0)),
        ],
        out_specs=pl.BlockSpec((_BB, _D), lambda i: (i, 0)),
        out_shape=jax.ShapeDtypeStruct((_B, _D), jnp.float32),
        compiler_params=pltpu.CompilerParams(
            dimension_semantics=("arbitrary",),
        ),
    )(x2, m2)


# speedup vs baseline: 2.7926x; 1.0609x over previous
import jax
import jax.numpy as jnp
from jax.experimental import pallas as pl
from jax.experimental.pallas import tpu as pltpu

_B, _L, _D = 16384, 200, 32
_BB = 256
_W = _L * _D
_WA = 128 * _D
_WB = _W - _WA


def _body(x_ref, m_ref, o_ref):
    m = m_ref[...]
    ma = m[:, 0:128]
    mb = m[:, 128:256]
    lane = jnp.arange(128, dtype=jnp.int32) // _D
    accs = [None, None, None, None]
    for k in range(50):
        src = ma if k < 32 else mb
        base = 4 * k if k < 32 else 4 * k - 128
        idx = jnp.broadcast_to((base + lane)[None, :], (_BB, 128))
        me = jnp.take_along_axis(src, idx, axis=1)
        term = x_ref[:, 128 * k:128 * (k + 1)] * me
        j = k % 4
        accs[j] = term if accs[j] is None else accs[j] + term
    acc = (accs[0] + accs[1]) + (accs[2] + accs[3])
    o_ref[...] = (acc[:, 0:32] + acc[:, 32:64]
                  + acc[:, 64:96] + acc[:, 96:128])


def kernel(inputs, mask):
    x2 = jnp.reshape(inputs, (_B, _W))
    m2 = jnp.pad(mask.astype(jnp.float32), ((0, 0), (0, 56)))
    return pl.pallas_call(
        _body,
        grid=(_B // _BB,),
        in_specs=[
            pl.BlockSpec((_BB, _W), lambda i: (i, 0)),
            pl.BlockSpec((_BB, 256), lambda i: (i, 0)),
        ],
        out_specs=pl.BlockSpec((_BB, _D), lambda i: (i, 0)),
        out_shape=jax.ShapeDtypeStruct((_B, _D), jnp.float32),
        compiler_params=pltpu.CompilerParams(
            dimension_semantics=("parallel",),
        ),
    )(x2, m2)
